# Initial kernel scaffold; baseline (speedup 1.0000x reference)
import jax
import jax.numpy as jnp
from jax.experimental import pallas as pl


def kernel(boxes, classification):
    raise NotImplementedError("write your pallas kernel here")


# speedup vs baseline: 11.7279x; 11.7279x over previous
import jax
import jax.numpy as jnp
from jax import lax
from jax.experimental import pallas as pl
from jax.experimental.pallas import tpu as pltpu

_NMS_T = 0.5
_SCORE_T = 0.01
_MAXDET = 100
_NEG_INF = float("-inf")


def _score_kernel(cls_ref, s_ref, l_ref):
    x = cls_ref[0]
    c = x.shape[1]
    m = jnp.max(x, axis=1)
    ci = lax.broadcasted_iota(jnp.int32, x.shape, 1)
    lab = jnp.min(jnp.where(x == m[:, None], ci, c), axis=1)
    s_ref[0, 0, :] = m
    l_ref[0, 0, :] = lab


def _nms_kernel(s_ref, x1_ref, y1_ref, x2_ref, y2_ref, lab_ref,
                os_ref, ox1_ref, oy1_ref, ox2_ref, oy2_ref, ol_ref,
                cur_ref, a2_ref):
    B, N = s_ref.shape
    iota = lax.broadcasted_iota(jnp.int32, (B, N), 1)
    oiota = lax.broadcasted_iota(jnp.int32, (B, _MAXDET), 1)

    s = s_ref[:, :]
    cur_ref[:, :] = jnp.where(s > _SCORE_T, s, _NEG_INF)
    a2_ref[:, :] = (x2_ref[:, :] - x1_ref[:, :]) * (y2_ref[:, :] - y1_ref[:, :])

    os_ref[:, :] = jnp.full((B, _MAXDET), -1.0, jnp.float32)
    ox1_ref[:, :] = jnp.full((B, _MAXDET), -1.0, jnp.float32)
    oy1_ref[:, :] = jnp.full((B, _MAXDET), -1.0, jnp.float32)
    ox2_ref[:, :] = jnp.full((B, _MAXDET), -1.0, jnp.float32)
    oy2_ref[:, :] = jnp.full((B, _MAXDET), -1.0, jnp.float32)
    ol_ref[:, :] = jnp.full((B, _MAXDET), -1, jnp.int32)

    def step(i, _):
        cur = cur_ref[:, :]
        m = jnp.max(cur, axis=1, keepdims=True)
        hit = cur == m
        idx = jnp.min(jnp.where(hit, iota, N), axis=1, keepdims=True)
        one = iota == idx

        X1 = x1_ref[:, :]
        Y1 = y1_ref[:, :]
        X2 = x2_ref[:, :]
        Y2 = y2_ref[:, :]
        bx1 = jnp.sum(jnp.where(one, X1, 0.0), axis=1, keepdims=True)
        by1 = jnp.sum(jnp.where(one, Y1, 0.0), axis=1, keepdims=True)
        bx2 = jnp.sum(jnp.where(one, X2, 0.0), axis=1, keepdims=True)
        by2 = jnp.sum(jnp.where(one, Y2, 0.0), axis=1, keepdims=True)
        blab = jnp.sum(jnp.where(one, lab_ref[:, :], 0), axis=1, keepdims=True)

        xx1 = jnp.maximum(bx1, X1)
        yy1 = jnp.maximum(by1, Y1)
        xx2 = jnp.minimum(bx2, X2)
        yy2 = jnp.minimum(by2, Y2)
        inter = jnp.maximum(xx2 - xx1, 0.0) * jnp.maximum(yy2 - yy1, 0.0)
        a1 = (bx2 - bx1) * (by2 - by1)
        iou = inter / (a1 + a2_ref[:, :] - inter + 1e-8)
        sup = iou > _NMS_T
        cur_ref[:, :] = jnp.where(sup | one, _NEG_INF, cur)

        valid = m > _NEG_INF
        sel = oiota == i
        os_ref[:, :] = jnp.where(sel, jnp.where(valid, m, -1.0), os_ref[:, :])
        ox1_ref[:, :] = jnp.where(sel, jnp.where(valid, bx1, -1.0), ox1_ref[:, :])
        oy1_ref[:, :] = jnp.where(sel, jnp.where(valid, by1, -1.0), oy1_ref[:, :])
        ox2_ref[:, :] = jnp.where(sel, jnp.where(valid, bx2, -1.0), ox2_ref[:, :])
        oy2_ref[:, :] = jnp.where(sel, jnp.where(valid, by2, -1.0), oy2_ref[:, :])
        ol_ref[:, :] = jnp.where(sel, jnp.where(valid, blab, -1), ol_ref[:, :])
        return 0

    lax.fori_loop(0, _MAXDET, step, 0)


def _scores_labels(classification):
    B, N, C = classification.shape
    chunk = 2000
    if N % chunk != 0:
        chunk = N
    nb = N // chunk
    s_flat, l_flat = pl.pallas_call(
        _score_kernel,
        grid=(B, nb),
        in_specs=[pl.BlockSpec((1, chunk, C), lambda b, j: (b, j, 0))],
        out_specs=[
            pl.BlockSpec((1, 1, chunk), lambda b, j, nb=nb: (b * nb + j, 0, 0)),
            pl.BlockSpec((1, 1, chunk), lambda b, j, nb=nb: (b * nb + j, 0, 0)),
        ],
        out_shape=[
            jax.ShapeDtypeStruct((B * nb, 1, chunk), jnp.float32),
            jax.ShapeDtypeStruct((B * nb, 1, chunk), jnp.int32),
        ],
    )(classification)
    return s_flat.reshape(B, N), l_flat.reshape(B, N)


def kernel(boxes, classification):
    B, N, C = classification.shape
    scores, labels = _scores_labels(classification)
    x1 = boxes[..., 0]
    y1 = boxes[..., 1]
    x2 = boxes[..., 2]
    y2 = boxes[..., 3]

    outs = pl.pallas_call(
        _nms_kernel,
        out_shape=[
            jax.ShapeDtypeStruct((B, _MAXDET), jnp.float32),
            jax.ShapeDtypeStruct((B, _MAXDET), jnp.float32),
            jax.ShapeDtypeStruct((B, _MAXDET), jnp.float32),
            jax.ShapeDtypeStruct((B, _MAXDET), jnp.float32),
            jax.ShapeDtypeStruct((B, _MAXDET), jnp.float32),
            jax.ShapeDtypeStruct((B, _MAXDET), jnp.int32),
        ],
        scratch_shapes=[
            pltpu.VMEM((B, N), jnp.float32),
            pltpu.VMEM((B, N), jnp.float32),
        ],
    )(scores, x1, y1, x2, y2, labels)
    os, ox1, oy1, ox2, oy2, ol = outs
    out_boxes = jnp.stack([ox1, oy1, ox2, oy2], axis=-1)
    return (out_boxes, os, ol)


# speedup vs baseline: 25.2726x; 2.1549x over previous
import jax
import jax.numpy as jnp
from jax import lax
from jax.experimental import pallas as pl
from jax.experimental.pallas import tpu as pltpu

_NMS_T = 0.5
_SCORE_T = 0.01
_MAXDET = 100
_NEG_INF = float("-inf")


def _score_kernel(cls_ref, s_ref, l_ref):
    x = cls_ref[0]
    c = x.shape[0]
    m = jnp.max(x, axis=0)
    ci = lax.broadcasted_iota(jnp.int32, x.shape, 0)
    lab = jnp.min(jnp.where(x == m[None, :], ci, c), axis=0)
    s_ref[0, 0, :] = m
    l_ref[0, 0, :] = lab


def _nms_kernel(s_ref, x1_ref, y1_ref, x2_ref, y2_ref, lab_ref,
                os_ref, ox1_ref, oy1_ref, ox2_ref, oy2_ref, ol_ref,
                cur_ref, a2_ref):
    B, N = s_ref.shape
    iota = lax.broadcasted_iota(jnp.int32, (B, N), 1)
    oiota = lax.broadcasted_iota(jnp.int32, (B, _MAXDET), 1)

    s = s_ref[:, :]
    cur_ref[:, :] = jnp.where(s > _SCORE_T, s, _NEG_INF)
    a2_ref[:, :] = (x2_ref[:, :] - x1_ref[:, :]) * (y2_ref[:, :] - y1_ref[:, :])

    os_ref[:, :] = jnp.full((B, _MAXDET), -1.0, jnp.float32)
    ox1_ref[:, :] = jnp.full((B, _MAXDET), -1.0, jnp.float32)
    oy1_ref[:, :] = jnp.full((B, _MAXDET), -1.0, jnp.float32)
    ox2_ref[:, :] = jnp.full((B, _MAXDET), -1.0, jnp.float32)
    oy2_ref[:, :] = jnp.full((B, _MAXDET), -1.0, jnp.float32)
    ol_ref[:, :] = jnp.full((B, _MAXDET), -1, jnp.int32)

    def step(i, _):
        cur = cur_ref[:, :]
        m = jnp.max(cur, axis=1, keepdims=True)
        hit = cur == m
        idx = jnp.min(jnp.where(hit, iota, N), axis=1, keepdims=True)
        one = iota == idx

        X1 = x1_ref[:, :]
        Y1 = y1_ref[:, :]
        X2 = x2_ref[:, :]
        Y2 = y2_ref[:, :]
        bx1 = jnp.sum(jnp.where(one, X1, 0.0), axis=1, keepdims=True)
        by1 = jnp.sum(jnp.where(one, Y1, 0.0), axis=1, keepdims=True)
        bx2 = jnp.sum(jnp.where(one, X2, 0.0), axis=1, keepdims=True)
        by2 = jnp.sum(jnp.where(one, Y2, 0.0), axis=1, keepdims=True)
        blab = jnp.sum(jnp.where(one, lab_ref[:, :], 0), axis=1, keepdims=True)

        xx1 = jnp.maximum(bx1, X1)
        yy1 = jnp.maximum(by1, Y1)
        xx2 = jnp.minimum(bx2, X2)
        yy2 = jnp.minimum(by2, Y2)
        inter = jnp.maximum(xx2 - xx1, 0.0) * jnp.maximum(yy2 - yy1, 0.0)
        a1 = (bx2 - bx1) * (by2 - by1)
        iou = inter / (a1 + a2_ref[:, :] - inter + 1e-8)
        sup = iou > _NMS_T
        cur_ref[:, :] = jnp.where(sup | one, _NEG_INF, cur)

        valid = m > _NEG_INF
        sel = oiota == i
        os_ref[:, :] = jnp.where(sel, jnp.where(valid, m, -1.0), os_ref[:, :])
        ox1_ref[:, :] = jnp.where(sel, jnp.where(valid, bx1, -1.0), ox1_ref[:, :])
        oy1_ref[:, :] = jnp.where(sel, jnp.where(valid, by1, -1.0), oy1_ref[:, :])
        ox2_ref[:, :] = jnp.where(sel, jnp.where(valid, bx2, -1.0), ox2_ref[:, :])
        oy2_ref[:, :] = jnp.where(sel, jnp.where(valid, by2, -1.0), oy2_ref[:, :])
        ol_ref[:, :] = jnp.where(sel, jnp.where(valid, blab, -1), ol_ref[:, :])
        return 0

    lax.fori_loop(0, _MAXDET, step, 0)


def _scores_labels(classification):
    B, N, C = classification.shape
    cls_t = jnp.transpose(classification, (0, 2, 1))
    s_flat, l_flat = pl.pallas_call(
        _score_kernel,
        grid=(B,),
        in_specs=[pl.BlockSpec((1, C, N), lambda b: (b, 0, 0))],
        out_specs=[
            pl.BlockSpec((1, 1, N), lambda b: (b, 0, 0)),
            pl.BlockSpec((1, 1, N), lambda b: (b, 0, 0)),
        ],
        out_shape=[
            jax.ShapeDtypeStruct((B, 1, N), jnp.float32),
            jax.ShapeDtypeStruct((B, 1, N), jnp.int32),
        ],
    )(cls_t)
    return s_flat.reshape(B, N), l_flat.reshape(B, N)


def kernel(boxes, classification):
    B, N, C = classification.shape
    scores, labels = _scores_labels(classification)
    x1 = boxes[..., 0]
    y1 = boxes[..., 1]
    x2 = boxes[..., 2]
    y2 = boxes[..., 3]

    outs = pl.pallas_call(
        _nms_kernel,
        out_shape=[
            jax.ShapeDtypeStruct((B, _MAXDET), jnp.float32),
            jax.ShapeDtypeStruct((B, _MAXDET), jnp.float32),
            jax.ShapeDtypeStruct((B, _MAXDET), jnp.float32),
            jax.ShapeDtypeStruct((B, _MAXDET), jnp.float32),
            jax.ShapeDtypeStruct((B, _MAXDET), jnp.float32),
            jax.ShapeDtypeStruct((B, _MAXDET), jnp.int32),
        ],
        scratch_shapes=[
            pltpu.VMEM((B, N), jnp.float32),
            pltpu.VMEM((B, N), jnp.float32),
        ],
    )(scores, x1, y1, x2, y2, labels)
    os, ox1, oy1, ox2, oy2, ol = outs
    out_boxes = jnp.stack([ox1, oy1, ox2, oy2], axis=-1)
    return (out_boxes, os, ol)
